# baseline (speedup 1.0000x reference)
import functools

import jax
import jax.numpy as jnp
from jax import lax
from jax.experimental import pallas as pl
from jax.experimental.pallas import tpu as pltpu
from jax.experimental.pallas import tpu_sc as plsc

N_ATOMS = 1048576
N_EMB = 128

NC = 2
NS = 16
NW = NC * NS

G = N_ATOMS // 128
GPW = G // NW
OUTER = GPW // 32
U1 = 28
CHUNK = 16
NCHUNK = GPW // CHUNK
NIBUF = 4
NBUF_D = 4
NBUF_V = 2
P_TOT = OUTER * U1
V_AT = {6: 0, 13: 1, 20: 2, 27: 3}


def _sc_gather(table, x2d):
    mesh = plsc.VectorSubcoreMesh(core_axis_name="c", subcore_axis_name="s")

    @functools.partial(
        pl.kernel,
        mesh=mesh,
        out_type=jax.ShapeDtypeStruct((G, 128, N_EMB), jnp.float32),
        scratch_types=[
            pltpu.VMEM((NIBUF, CHUNK, 128), jnp.int32),
            pltpu.VMEM((NBUF_D, 128, N_EMB), jnp.float32),
            pltpu.VMEM((NBUF_V, 128, N_EMB), jnp.float32),
            pltpu.VMEM((100, N_EMB), jnp.float32),
            pltpu.VMEM_SHARED((100, N_EMB), jnp.float32),
            pltpu.SemaphoreType.DMA,
            pltpu.SemaphoreType.DMA,
            pltpu.SemaphoreType.DMA,
            pltpu.SemaphoreType.DMA,
        ],
    )
    def body(table_hbm, idx_hbm, out_hbm, idx_c, rows_d, rows_v2, table_loc,
             table_sh, sem_gd, sem_sd, sem_sv, sem_i):
        wid = lax.axis_index("s") * NC + lax.axis_index("c")
        base = wid * GPW

        @pl.when(lax.axis_index("s") == 0)
        def _():
            pltpu.sync_copy(table_hbm, table_sh)

        pltpu.sync_copy(table_hbm, table_loc)
        pltpu.sync_copy(idx_hbm.at[pl.ds(base, CHUNK)], idx_c.at[0])
        pltpu.sync_copy(idx_hbm.at[pl.ds(base + CHUNK, CHUNK)], idx_c.at[1])
        plsc.subcore_barrier()

        def idx_ref(l):
            return idx_c.at[(l // CHUNK) % NIBUF, l % CHUNK]

        def idx_fetch(ck):
            pltpu.async_copy(idx_hbm.at[pl.ds(base + ck * CHUNK, CHUNK)],
                             idx_c.at[ck % NIBUF], sem_i)

        def idx_wait(ck):
            pltpu.make_async_copy(
                idx_hbm.at[pl.ds(base + ck * CHUNK, CHUNK)],
                idx_c.at[ck % NIBUF], sem_i).wait()

        def gd(l, b):
            pltpu.async_copy(table_sh.at[idx_ref(l)], rows_d.at[b], sem_gd)

        def gd_wait(l, b):
            pltpu.make_async_copy(
                table_sh.at[idx_ref(l)], rows_d.at[b], sem_gd).wait()

        def sd(l, b):
            pltpu.async_copy(rows_d.at[b], out_hbm.at[base + l], sem_sd)

        def sd_wait(l, b):
            pltpu.make_async_copy(
                rows_d.at[b], out_hbm.at[base + l], sem_sd).wait()

        def sv(l, b):
            pltpu.async_copy(rows_v2.at[b], out_hbm.at[base + l], sem_sv)

        def sv_wait(l, b):
            pltpu.make_async_copy(
                rows_v2.at[b], out_hbm.at[base + l], sem_sv).wait()

        def fill(l, b):
            lrow = idx_ref(l)

            def rbody(rb, carry):
                iv = lrow[pl.ds(rb * 16, 16)]
                ro = rb * 16
                for k in range(16):
                    r = iv[k]
                    for c in range(8):
                        rows_v2[b, ro + k, pl.ds(c * 16, 16)] = (
                            table_loc[r, pl.ds(c * 16, 16)])
                return carry

            lax.fori_loop(0, 8, rbody, 0)

        gd(0, 0)
        gd(1, 1)

        def step(i, carry):
            @pl.when(i <= OUTER - 2)
            def _():
                idx_fetch(2 * i + 2)
                idx_fetch(2 * i + 3)

            for u in range(U1):
                p = i * U1 + u
                l = i * 32 + u
                gd_wait(l, u % NBUF_D)
                sd(l, u % NBUF_D)

                @pl.when(p >= 2)
                def _():
                    sd_wait(l - 2 if u >= 2 else l - 6, (u + 2) % NBUF_D)

                if u == 24:
                    @pl.when(i <= OUTER - 2)
                    def _():
                        idx_wait(2 * i + 2)
                        idx_wait(2 * i + 3)

                @pl.when(p + 2 < P_TOT)
                def _():
                    gd(l + 2 if u + 2 < U1 else l + 6, (u + 2) % NBUF_D)

                if u in V_AT:
                    m = V_AT[u]
                    v = i * 4 + m
                    lv = i * 32 + U1 + m

                    @pl.when(v >= 2)
                    def _():
                        sv_wait(lv - 2 if m >= 2 else lv - 30, m % NBUF_V)

                    fill(lv, m % NBUF_V)
                    sv(lv, m % NBUF_V)

            return carry

        lax.fori_loop(0, OUTER, step, 0)

        last = (OUTER - 1) * 32
        sd_wait(last + U1 - 2, (U1 - 2) % NBUF_D)
        sd_wait(last + U1 - 1, (U1 - 1) % NBUF_D)
        sv_wait(last + U1 + 2, 0)
        sv_wait(last + U1 + 3, 1)

    return body(table, x2d)


def kernel(x, embedding_list):
    out = _sc_gather(embedding_list, x.reshape(G, 128))
    return out.reshape(N_ATOMS, N_EMB)


# speedup vs baseline: 1.1139x; 1.1139x over previous
import functools

import jax
import jax.numpy as jnp
from jax import lax
from jax.experimental import pallas as pl
from jax.experimental.pallas import tpu as pltpu
from jax.experimental.pallas import tpu_sc as plsc

N_ATOMS = 1048576
N_EMB = 128

NC = 2
NS = 16
NW = NC * NS

G = N_ATOMS // 128
GPW = G // NW
OUTER = GPW // 32
U1 = 28
CHUNK = 16
NCHUNK = GPW // CHUNK
NIBUF = 4
NBUF_D = 4
NBUF_V = 2
P_TOT = OUTER * U1
V_AT = {3: 0, 6: 1, 9: 2, 13: 3, 16: 4, 20: 5, 23: 6, 26: 7}


def _sc_gather(table, x2d):
    mesh = plsc.VectorSubcoreMesh(core_axis_name="c", subcore_axis_name="s")

    @functools.partial(
        pl.kernel,
        mesh=mesh,
        out_type=jax.ShapeDtypeStruct((G, 128, N_EMB), jnp.float32),
        scratch_types=[
            pltpu.VMEM((NIBUF, CHUNK, 128), jnp.int32),
            pltpu.VMEM((NBUF_D, 128, N_EMB), jnp.float32),
            pltpu.VMEM((NBUF_V, 128, N_EMB), jnp.float32),
            pltpu.VMEM((100, N_EMB), jnp.float32),
            pltpu.VMEM_SHARED((100, N_EMB), jnp.float32),
            pltpu.SemaphoreType.DMA,
            pltpu.SemaphoreType.DMA,
            pltpu.SemaphoreType.DMA,
            pltpu.SemaphoreType.DMA,
        ],
    )
    def body(table_hbm, idx_hbm, out_hbm, idx_c, rows_d, rows_v2, table_loc,
             table_sh, sem_gd, sem_sd, sem_sv, sem_i):
        wid = lax.axis_index("s") * NC + lax.axis_index("c")
        base = wid * GPW

        @pl.when(lax.axis_index("s") == 0)
        def _():
            pltpu.sync_copy(table_hbm, table_sh)

        pltpu.sync_copy(table_hbm, table_loc)
        pltpu.sync_copy(idx_hbm.at[pl.ds(base, CHUNK)], idx_c.at[0])
        pltpu.sync_copy(idx_hbm.at[pl.ds(base + CHUNK, CHUNK)], idx_c.at[1])
        plsc.subcore_barrier()

        def idx_ref(l):
            return idx_c.at[(l // CHUNK) % NIBUF, l % CHUNK]

        def idx_fetch(ck):
            pltpu.async_copy(idx_hbm.at[pl.ds(base + ck * CHUNK, CHUNK)],
                             idx_c.at[ck % NIBUF], sem_i)

        def idx_wait(ck):
            pltpu.make_async_copy(
                idx_hbm.at[pl.ds(base + ck * CHUNK, CHUNK)],
                idx_c.at[ck % NIBUF], sem_i).wait()

        def gd(l, b):
            pltpu.async_copy(table_sh.at[idx_ref(l)], rows_d.at[b], sem_gd)

        def gd_wait(l, b):
            pltpu.make_async_copy(
                table_sh.at[idx_ref(l)], rows_d.at[b], sem_gd).wait()

        def sd(l, b):
            pltpu.async_copy(rows_d.at[b], out_hbm.at[base + l], sem_sd)

        def sd_wait(l, b):
            pltpu.make_async_copy(
                rows_d.at[b], out_hbm.at[base + l], sem_sd).wait()

        def sv(l, b):
            pltpu.async_copy(rows_v2.at[b], out_hbm.at[base + l], sem_sv)

        def sv_wait(l, b):
            pltpu.make_async_copy(
                rows_v2.at[b], out_hbm.at[base + l], sem_sv).wait()

        def fill_half(l, b, h):
            lrow = idx_ref(l)

            def rbody(rbq, carry):
                rb = h * 4 + rbq
                iv = lrow[pl.ds(rb * 16, 16)]
                ro = rb * 16
                for k in range(16):
                    r = iv[k]
                    for c in range(8):
                        rows_v2[b, ro + k, pl.ds(c * 16, 16)] = (
                            table_loc[r, pl.ds(c * 16, 16)])
                return carry

            lax.fori_loop(0, 4, rbody, 0)

        gd(0, 0)
        gd(1, 1)

        def step(i, carry):
            @pl.when(i <= OUTER - 2)
            def _():
                idx_fetch(2 * i + 2)
                idx_fetch(2 * i + 3)

            for u in range(U1):
                p = i * U1 + u
                l = i * 32 + u
                gd_wait(l, u % NBUF_D)
                sd(l, u % NBUF_D)

                @pl.when(p >= 2)
                def _():
                    sd_wait(l - 2 if u >= 2 else l - 6, (u + 2) % NBUF_D)

                if u == 24:
                    @pl.when(i <= OUTER - 2)
                    def _():
                        idx_wait(2 * i + 2)
                        idx_wait(2 * i + 3)

                @pl.when(p + 2 < P_TOT)
                def _():
                    gd(l + 2 if u + 2 < U1 else l + 6, (u + 2) % NBUF_D)

                if u in V_AT:
                    m2 = V_AT[u]
                    m, h = m2 // 2, m2 % 2
                    v = i * 4 + m
                    lv = i * 32 + U1 + m

                    if h == 0:
                        @pl.when(v >= 2)
                        def _():
                            sv_wait(lv - 2 if m >= 2 else lv - 30,
                                    m % NBUF_V)

                    fill_half(lv, m % NBUF_V, h)
                    if h == 1:
                        sv(lv, m % NBUF_V)

            return carry

        lax.fori_loop(0, OUTER, step, 0)

        last = (OUTER - 1) * 32
        sd_wait(last + U1 - 2, (U1 - 2) % NBUF_D)
        sd_wait(last + U1 - 1, (U1 - 1) % NBUF_D)
        sv_wait(last + U1 + 2, 0)
        sv_wait(last + U1 + 3, 1)

    return body(table, x2d)


def kernel(x, embedding_list):
    out = _sc_gather(embedding_list, x.reshape(G, 128))
    return out.reshape(N_ATOMS, N_EMB)


# speedup vs baseline: 1.2645x; 1.1352x over previous
import functools

import jax
import jax.numpy as jnp
from jax import lax
from jax.experimental import pallas as pl
from jax.experimental.pallas import tpu as pltpu
from jax.experimental.pallas import tpu_sc as plsc

N_ATOMS = 1048576
N_EMB = 128

NC = 2
NS = 16
NW = NC * NS

G = N_ATOMS // 128
GPW = G // NW
OUTER = GPW // 32
U1 = 28
CHUNK = 16
NCHUNK = GPW // CHUNK
NIBUF = 4
NBUF_D = 4
NBUF_V = 2
P_TOT = OUTER * U1
V_AT = {2: 0, 4: 1, 5: 2, 7: 3, 9: 4, 10: 5, 12: 6, 14: 7,
        15: 8, 17: 9, 19: 10, 20: 11, 22: 12, 24: 13, 25: 14, 27: 15}


def _sc_gather(table, x2d):
    mesh = plsc.VectorSubcoreMesh(core_axis_name="c", subcore_axis_name="s")

    @functools.partial(
        pl.kernel,
        mesh=mesh,
        out_type=jax.ShapeDtypeStruct((G, 128, N_EMB), jnp.float32),
        scratch_types=[
            pltpu.VMEM((NIBUF, CHUNK, 128), jnp.int32),
            pltpu.VMEM((NBUF_D, 128, N_EMB), jnp.float32),
            pltpu.VMEM((NBUF_V, 128, N_EMB), jnp.float32),
            pltpu.VMEM((100, N_EMB), jnp.float32),
            pltpu.VMEM_SHARED((100, N_EMB), jnp.float32),
            pltpu.SemaphoreType.DMA,
            pltpu.SemaphoreType.DMA,
            pltpu.SemaphoreType.DMA,
            pltpu.SemaphoreType.DMA,
        ],
    )
    def body(table_hbm, idx_hbm, out_hbm, idx_c, rows_d, rows_v2, table_loc,
             table_sh, sem_gd, sem_sd, sem_sv, sem_i):
        wid = lax.axis_index("s") * NC + lax.axis_index("c")
        base = wid * GPW

        @pl.when(lax.axis_index("s") == 0)
        def _():
            pltpu.sync_copy(table_hbm, table_sh)

        pltpu.sync_copy(table_hbm, table_loc)
        pltpu.sync_copy(idx_hbm.at[pl.ds(base, CHUNK)], idx_c.at[0])
        pltpu.sync_copy(idx_hbm.at[pl.ds(base + CHUNK, CHUNK)], idx_c.at[1])
        plsc.subcore_barrier()

        def idx_ref(l):
            return idx_c.at[(l // CHUNK) % NIBUF, l % CHUNK]

        def idx_fetch(ck):
            pltpu.async_copy(idx_hbm.at[pl.ds(base + ck * CHUNK, CHUNK)],
                             idx_c.at[ck % NIBUF], sem_i)

        def idx_wait(ck):
            pltpu.make_async_copy(
                idx_hbm.at[pl.ds(base + ck * CHUNK, CHUNK)],
                idx_c.at[ck % NIBUF], sem_i).wait()

        def gd(l, b):
            pltpu.async_copy(table_sh.at[idx_ref(l)], rows_d.at[b], sem_gd)

        def gd_wait(l, b):
            pltpu.make_async_copy(
                table_sh.at[idx_ref(l)], rows_d.at[b], sem_gd).wait()

        def sd(l, b):
            pltpu.async_copy(rows_d.at[b], out_hbm.at[base + l], sem_sd)

        def sd_wait(l, b):
            pltpu.make_async_copy(
                rows_d.at[b], out_hbm.at[base + l], sem_sd).wait()

        def sv(l, b):
            pltpu.async_copy(rows_v2.at[b], out_hbm.at[base + l], sem_sv)

        def sv_wait(l, b):
            pltpu.make_async_copy(
                rows_v2.at[b], out_hbm.at[base + l], sem_sv).wait()

        def fill_quarter(l, b, h):
            lrow = idx_ref(l)

            def rbody(rbq, carry):
                rb = h * 2 + rbq
                iv = lrow[pl.ds(rb * 16, 16)]
                ro = rb * 16
                for k in range(16):
                    r = iv[k]
                    for c in range(8):
                        rows_v2[b, ro + k, pl.ds(c * 16, 16)] = (
                            table_loc[r, pl.ds(c * 16, 16)])
                return carry

            lax.fori_loop(0, 2, rbody, 0)

        gd(0, 0)
        gd(1, 1)

        def step(i, carry):
            @pl.when(i <= OUTER - 2)
            def _():
                idx_fetch(2 * i + 2)
                idx_fetch(2 * i + 3)

            for u in range(U1):
                p = i * U1 + u
                l = i * 32 + u
                gd_wait(l, u % NBUF_D)
                sd(l, u % NBUF_D)

                @pl.when(p >= 2)
                def _():
                    sd_wait(l - 2 if u >= 2 else l - 6, (u + 2) % NBUF_D)

                if u == 24:
                    @pl.when(i <= OUTER - 2)
                    def _():
                        idx_wait(2 * i + 2)
                        idx_wait(2 * i + 3)

                @pl.when(p + 2 < P_TOT)
                def _():
                    gd(l + 2 if u + 2 < U1 else l + 6, (u + 2) % NBUF_D)

                if u in V_AT:
                    m4 = V_AT[u]
                    m, h = m4 // 4, m4 % 4
                    v = i * 4 + m
                    lv = i * 32 + U1 + m

                    if h == 0:
                        @pl.when(v >= 2)
                        def _():
                            sv_wait(lv - 2 if m >= 2 else lv - 30,
                                    m % NBUF_V)

                    fill_quarter(lv, m % NBUF_V, h)
                    if h == 3:
                        sv(lv, m % NBUF_V)

            return carry

        lax.fori_loop(0, OUTER, step, 0)

        last = (OUTER - 1) * 32
        sd_wait(last + U1 - 2, (U1 - 2) % NBUF_D)
        sd_wait(last + U1 - 1, (U1 - 1) % NBUF_D)
        sv_wait(last + U1 + 2, 0)
        sv_wait(last + U1 + 3, 1)

    return body(table, x2d)


def kernel(x, embedding_list):
    out = _sc_gather(embedding_list, x.reshape(G, 128))
    return out.reshape(N_ATOMS, N_EMB)
